# baseline (speedup 1.0000x reference)
import functools
import math

import jax
import jax.numpy as jnp
from jax import lax
from jax.experimental import pallas as pl
from jax.experimental.pallas import tpu as pltpu
from jax.experimental.pallas import tpu_sc as plsc

B, L, D = 1024, 200, 256
N = B * L
NC, NS, LANES = 2, 16, 16
NW = NC * NS
PER_W = N // NW
C = 128
NCHUNK = PER_W // C
SCALE = 16.0

_mesh = plsc.VectorSubcoreMesh(core_axis_name="c", subcore_axis_name="s")


@functools.partial(
    pl.kernel,
    mesh=_mesh,
    out_type=jax.ShapeDtypeStruct((N, D), jnp.float32),
    scratch_types=[
        pltpu.VMEM((C,), jnp.int32),
        pltpu.VMEM((C,), jnp.int32),
        pltpu.VMEM((C,), jnp.int32),
        pltpu.VMEM((C, D), jnp.float32),
        pltpu.VMEM((C, D), jnp.float32),
        pltpu.VMEM((C, D), jnp.float32),
        pltpu.SemaphoreType.DMA,
    ],
)
def _embed_sum(vt, ct, pt, val_tab, coord_tab, pos_tab, out,
               idx_v, idx_c, idx_p, rows_a, rows_b, rows_c, sem):
    wid = lax.axis_index("s") * NC + lax.axis_index("c")
    base = wid * PER_W

    def chunk(i, carry):
        off = base + i * C
        pltpu.sync_copy(vt.at[pl.ds(off, C)], idx_v)
        pltpu.sync_copy(ct.at[pl.ds(off, C)], idx_c)
        pltpu.sync_copy(pt.at[pl.ds(off, C)], idx_p)
        pltpu.async_copy(val_tab.at[idx_v], rows_a, sem).wait()
        pltpu.async_copy(coord_tab.at[idx_c], rows_b, sem).wait()
        pltpu.async_copy(pos_tab.at[idx_p], rows_c, sem).wait()

        def tok(t, c2):
            for j in range(D // LANES):
                s = pl.ds(j * LANES, LANES)
                rows_a[t, s] = (rows_a[t, s] + rows_b[t, s] + rows_c[t, s]) * SCALE
            return c2

        lax.fori_loop(0, C, tok, 0)
        pltpu.sync_copy(rows_a, out.at[pl.ds(off, C)])
        return carry

    lax.fori_loop(0, NCHUNK, chunk, 0)


def kernel(value_tokens, coord_type_tokens, position_tokens,
           value_table, coord_type_table, position_table):
    vt = value_tokens.reshape(N).astype(jnp.int32)
    ct = coord_type_tokens.reshape(N).astype(jnp.int32)
    pt = position_tokens.reshape(N).astype(jnp.int32)
    out = _embed_sum(vt, ct, pt, value_table, coord_type_table, position_table)
    return out.reshape(B, L, D)
